# baseline (speedup 1.0000x reference)
import functools

import jax
import jax.numpy as jnp
from jax import lax
from jax.experimental import pallas as pl
from jax.experimental.pallas import tpu as pltpu
from jax.experimental.pallas import tpu_sc as plsc

N = 50000
D = 256
G = 128
NC = 2
NS = 16
NW = NC * NS
S = 80
L = 16

NSC = 17920
RTC = N - NSC
C = NSC // NW
NIT = C // S

B = 2048
NB_TC = -(-RTC // B)


def _phase1_body(x_hbm, ids_hbm, zeros_hbm, part_hbm,
                 buf_a, buf_b, buf_c, idx_a, idx_b, idx_c, acc, rbuf, slab,
                 sem_a, sem_b, sem_c, sem_ia, sem_ib, sem_ic, sem_z):
    cid = lax.axis_index("c")
    sid = lax.axis_index("s")
    wid = sid * NC + cid
    base = RTC + wid * C

    def start(i, buf, ibuf, rsem, isem):
        off = base + i * S
        pltpu.make_async_copy(x_hbm.at[pl.ds(off, S)], buf, rsem).start()
        pltpu.make_async_copy(ids_hbm.at[pl.ds(off, S)], ibuf, isem).start()

    start(0, buf_a, idx_a, sem_a, sem_ia)
    start(1, buf_b, idx_b, sem_b, sem_ib)
    start(2, buf_c, idx_c, sem_c, sem_ic)
    pltpu.make_async_copy(zeros_hbm, acc, sem_z).start()
    pltpu.make_async_copy(zeros_hbm, acc, sem_z).wait()

    bufs = ((buf_a, idx_a, sem_a, sem_ia), (buf_b, idx_b, sem_b, sem_ib),
            (buf_c, idx_c, sem_c, sem_ic))
    NB = len(bufs)

    def body(k, carry):
        for b in range(NB):
            buf, ibuf, rsem, isem = bufs[b]
            i = NB * k + b

            @pl.when(i < NIT)
            def _():
                off = base + i * S
                pltpu.make_async_copy(
                    x_hbm.at[pl.ds(off, S)], buf, rsem).wait()
                pltpu.make_async_copy(
                    ids_hbm.at[pl.ds(off, S)], ibuf, isem).wait()

                def group_body(g, gcarry):
                    idv = ibuf[pl.ds(g * L, L)]
                    seg0 = idv[0]
                    uniform = seg0 == idv[L - 1]

                    @pl.when(uniform)
                    def _():
                        gbase = seg0 * D
                        for j in range(D // L):
                            vs = [buf[g * L + r, pl.ds(L * j, L)]
                                  for r in range(L)]
                            while len(vs) > 1:
                                vs = [vs[t] + vs[t + 1]
                                      for t in range(0, len(vs), 2)]
                            plsc.addupdate(acc.at[pl.ds(gbase + L * j, L)],
                                           vs[0])

                    @pl.when(jnp.logical_not(uniform))
                    def _():
                        for r in range(L):
                            rbase = idv[r] * D
                            for j in range(D // L):
                                v = buf[g * L + r, pl.ds(L * j, L)]
                                plsc.addupdate(
                                    acc.at[pl.ds(rbase + L * j, L)], v)

                    return gcarry

                lax.fori_loop(0, S // L, group_body, 0)

                @pl.when(i + NB < NIT)
                def _():
                    start(i + NB, buf, ibuf, rsem, isem)
        return carry

    lax.fori_loop(0, (NIT + NB - 1) // NB, body, 0)

    R = 4
    GR = G // R
    W = (GR // NS) * D
    for h in range(R):
        pltpu.sync_copy(acc.at[pl.ds(h * GR * D, GR * D)],
                        slab.at[pl.ds(sid * GR * D, GR * D)])
        plsc.subcore_barrier()
        for s in range(NS):
            pltpu.sync_copy(slab.at[pl.ds(s * GR * D + sid * W, W)],
                            rbuf.at[pl.ds(s * W, W)])

        def red_body(w, rcarry):
            vs = [rbuf[pl.ds(s * W + w * L, L)] for s in range(NS)]
            while len(vs) > 1:
                vs = [vs[t] + vs[t + 1] for t in range(0, len(vs), 2)]
            rbuf[pl.ds(w * L, L)] = vs[0]
            return rcarry

        lax.fori_loop(0, W // L, red_body, 0)
        pltpu.sync_copy(
            rbuf.at[pl.ds(0, W)],
            part_hbm.at[pl.ds(cid * G * D + h * GR * D + sid * W, W)])
        plsc.subcore_barrier()


_phase1 = functools.partial(
    pl.kernel,
    out_type=jax.ShapeDtypeStruct((NC * G * D,), jnp.float32),
    mesh=plsc.VectorSubcoreMesh(core_axis_name="c", subcore_axis_name="s"),
    scratch_types=[
        pltpu.VMEM((S, D), jnp.float32),
        pltpu.VMEM((S, D), jnp.float32),
        pltpu.VMEM((S, D), jnp.float32),
        pltpu.VMEM((S,), jnp.int32),
        pltpu.VMEM((S,), jnp.int32),
        pltpu.VMEM((S,), jnp.int32),
        pltpu.VMEM((G * D,), jnp.float32),
        pltpu.VMEM((NS * (G // 4 // NS) * D,), jnp.float32),
        pltpu.MemorySpace.VMEM_SHARED((NS * (G // 4) * D,), jnp.float32),
        pltpu.SemaphoreType.DMA,
        pltpu.SemaphoreType.DMA,
        pltpu.SemaphoreType.DMA,
        pltpu.SemaphoreType.DMA,
        pltpu.SemaphoreType.DMA,
        pltpu.SemaphoreType.DMA,
        pltpu.SemaphoreType.DMA,
    ],
)(_phase1_body)


def _tc_body(x_ref, ids_ref, o_ref):
    ib = pl.program_id(0)

    @pl.when(ib == 0)
    def _():
        o_ref[...] = jnp.zeros((G, D), jnp.float32)

    ids_b = ids_ref[0, 0, :]
    onehot = (ids_b[:, None]
              == lax.broadcasted_iota(jnp.int32, (B, G), 1)).astype(jnp.bfloat16)
    x = x_ref[...]
    x_hi = x.astype(jnp.bfloat16)
    x_lo = (x - x_hi.astype(jnp.float32)).astype(jnp.bfloat16)
    dn = (((0,), (0,)), ((), ()))
    o_ref[...] += (
        lax.dot_general(onehot, x_hi, dn, preferred_element_type=jnp.float32)
        + lax.dot_general(onehot, x_lo, dn, preferred_element_type=jnp.float32))


def _tc_partial(x, ids_tc):
    return pl.pallas_call(
        _tc_body,
        grid=(NB_TC,),
        in_specs=[
            pl.BlockSpec((B, D), lambda ib: (ib, 0)),
            pl.BlockSpec((1, 1, B), lambda ib: (ib, 0, 0)),
        ],
        out_specs=pl.BlockSpec((G, D), lambda ib: (0, 0)),
        out_shape=jax.ShapeDtypeStruct((G, D), jnp.float32),
    )(x, ids_tc)


def _combine_body(p_ref, t_ref, o_ref):
    o_ref[...] = p_ref[0] + p_ref[1] + t_ref[...]


def _combine(partials_sc, partial_tc):
    return pl.pallas_call(
        _combine_body,
        out_shape=jax.ShapeDtypeStruct((G, D), jnp.float32),
    )(partials_sc, partial_tc)


@jax.jit
def kernel(node_feature, segment_ids, num_graphs):
    ids = segment_ids.astype(jnp.int32)
    zeros = jnp.zeros((G * D,), jnp.float32)
    ids_tc = jnp.concatenate(
        [ids[:RTC], jnp.full((NB_TC * B - RTC,), G, jnp.int32)]
    ).reshape(NB_TC, 1, B)
    partials_sc = _phase1(node_feature, ids, zeros)
    partial_tc = _tc_partial(node_feature, ids_tc)
    return _combine(partials_sc.reshape(NC, G, D), partial_tc)


# speedup vs baseline: 1.0960x; 1.0960x over previous
import functools

import jax
import jax.numpy as jnp
from jax import lax
from jax.experimental import pallas as pl
from jax.experimental.pallas import tpu as pltpu
from jax.experimental.pallas import tpu_sc as plsc

N = 50000
D = 256
G = 128
NC = 2
NS = 16
NW = NC * NS
S = 80
L = 16

NSC = 10240
RTC = N - NSC
C = NSC // NW
NIT = C // S

B = 2048
NB_TC = -(-RTC // B)


def _phase1_body(x_hbm, ids_hbm, zeros_hbm, part_hbm,
                 buf_a, buf_b, buf_c, idx_a, idx_b, idx_c, acc, rbuf, slab,
                 sem_a, sem_b, sem_c, sem_ia, sem_ib, sem_ic, sem_z):
    cid = lax.axis_index("c")
    sid = lax.axis_index("s")
    wid = sid * NC + cid
    base = RTC + wid * C

    def start(i, buf, ibuf, rsem, isem):
        off = base + i * S
        pltpu.make_async_copy(x_hbm.at[pl.ds(off, S)], buf, rsem).start()
        pltpu.make_async_copy(ids_hbm.at[pl.ds(off, S)], ibuf, isem).start()

    start(0, buf_a, idx_a, sem_a, sem_ia)
    start(1, buf_b, idx_b, sem_b, sem_ib)
    start(2, buf_c, idx_c, sem_c, sem_ic)
    pltpu.make_async_copy(zeros_hbm, acc, sem_z).start()
    pltpu.make_async_copy(zeros_hbm, acc, sem_z).wait()

    bufs = ((buf_a, idx_a, sem_a, sem_ia), (buf_b, idx_b, sem_b, sem_ib),
            (buf_c, idx_c, sem_c, sem_ic))
    NB = len(bufs)

    def body(k, carry):
        for b in range(NB):
            buf, ibuf, rsem, isem = bufs[b]
            i = NB * k + b

            @pl.when(i < NIT)
            def _():
                off = base + i * S
                pltpu.make_async_copy(
                    x_hbm.at[pl.ds(off, S)], buf, rsem).wait()
                pltpu.make_async_copy(
                    ids_hbm.at[pl.ds(off, S)], ibuf, isem).wait()

                def group_body(g, gcarry):
                    idv = ibuf[pl.ds(g * L, L)]
                    seg0 = idv[0]
                    uniform = seg0 == idv[L - 1]

                    @pl.when(uniform)
                    def _():
                        gbase = seg0 * D
                        for j in range(D // L):
                            vs = [buf[g * L + r, pl.ds(L * j, L)]
                                  for r in range(L)]
                            while len(vs) > 1:
                                vs = [vs[t] + vs[t + 1]
                                      for t in range(0, len(vs), 2)]
                            plsc.addupdate(acc.at[pl.ds(gbase + L * j, L)],
                                           vs[0])

                    @pl.when(jnp.logical_not(uniform))
                    def _():
                        for r in range(L):
                            rbase = idv[r] * D
                            for j in range(D // L):
                                v = buf[g * L + r, pl.ds(L * j, L)]
                                plsc.addupdate(
                                    acc.at[pl.ds(rbase + L * j, L)], v)

                    return gcarry

                lax.fori_loop(0, S // L, group_body, 0)

                @pl.when(i + NB < NIT)
                def _():
                    start(i + NB, buf, ibuf, rsem, isem)
        return carry

    lax.fori_loop(0, (NIT + NB - 1) // NB, body, 0)

    R = 4
    GR = G // R
    W = (GR // NS) * D
    for h in range(R):
        pltpu.sync_copy(acc.at[pl.ds(h * GR * D, GR * D)],
                        slab.at[pl.ds(sid * GR * D, GR * D)])
        plsc.subcore_barrier()
        for s in range(NS):
            pltpu.sync_copy(slab.at[pl.ds(s * GR * D + sid * W, W)],
                            rbuf.at[pl.ds(s * W, W)])

        def red_body(w, rcarry):
            vs = [rbuf[pl.ds(s * W + w * L, L)] for s in range(NS)]
            while len(vs) > 1:
                vs = [vs[t] + vs[t + 1] for t in range(0, len(vs), 2)]
            rbuf[pl.ds(w * L, L)] = vs[0]
            return rcarry

        lax.fori_loop(0, W // L, red_body, 0)
        pltpu.sync_copy(
            rbuf.at[pl.ds(0, W)],
            part_hbm.at[pl.ds(cid * G * D + h * GR * D + sid * W, W)])
        plsc.subcore_barrier()


_phase1 = functools.partial(
    pl.kernel,
    out_type=jax.ShapeDtypeStruct((NC * G * D,), jnp.float32),
    mesh=plsc.VectorSubcoreMesh(core_axis_name="c", subcore_axis_name="s"),
    scratch_types=[
        pltpu.VMEM((S, D), jnp.float32),
        pltpu.VMEM((S, D), jnp.float32),
        pltpu.VMEM((S, D), jnp.float32),
        pltpu.VMEM((S,), jnp.int32),
        pltpu.VMEM((S,), jnp.int32),
        pltpu.VMEM((S,), jnp.int32),
        pltpu.VMEM((G * D,), jnp.float32),
        pltpu.VMEM((NS * (G // 4 // NS) * D,), jnp.float32),
        pltpu.MemorySpace.VMEM_SHARED((NS * (G // 4) * D,), jnp.float32),
        pltpu.SemaphoreType.DMA,
        pltpu.SemaphoreType.DMA,
        pltpu.SemaphoreType.DMA,
        pltpu.SemaphoreType.DMA,
        pltpu.SemaphoreType.DMA,
        pltpu.SemaphoreType.DMA,
        pltpu.SemaphoreType.DMA,
    ],
)(_phase1_body)


def _tc_body(x_ref, ids_ref, o_ref):
    ib = pl.program_id(0)

    @pl.when(ib == 0)
    def _():
        o_ref[...] = jnp.zeros((G, D), jnp.float32)

    ids_b = ids_ref[0, 0, :]
    onehot = (ids_b[:, None]
              == lax.broadcasted_iota(jnp.int32, (B, G), 1)).astype(jnp.bfloat16)
    x = x_ref[...]
    x_hi = x.astype(jnp.bfloat16)
    x_lo = (x - x_hi.astype(jnp.float32)).astype(jnp.bfloat16)
    dn = (((0,), (0,)), ((), ()))
    o_ref[...] += (
        lax.dot_general(onehot, x_hi, dn, preferred_element_type=jnp.float32)
        + lax.dot_general(onehot, x_lo, dn, preferred_element_type=jnp.float32))


def _tc_partial(x, ids_tc):
    return pl.pallas_call(
        _tc_body,
        grid=(NB_TC,),
        in_specs=[
            pl.BlockSpec((B, D), lambda ib: (ib, 0)),
            pl.BlockSpec((1, 1, B), lambda ib: (ib, 0, 0)),
        ],
        out_specs=pl.BlockSpec((G, D), lambda ib: (0, 0)),
        out_shape=jax.ShapeDtypeStruct((G, D), jnp.float32),
    )(x, ids_tc)


def _combine_body(p_ref, t_ref, o_ref):
    o_ref[...] = p_ref[0] + p_ref[1] + t_ref[...]


def _combine(partials_sc, partial_tc):
    return pl.pallas_call(
        _combine_body,
        out_shape=jax.ShapeDtypeStruct((G, D), jnp.float32),
    )(partials_sc, partial_tc)


@jax.jit
def kernel(node_feature, segment_ids, num_graphs):
    ids = segment_ids.astype(jnp.int32)
    zeros = jnp.zeros((G * D,), jnp.float32)
    ids_tc = jnp.concatenate(
        [ids[:RTC], jnp.full((NB_TC * B - RTC,), G, jnp.int32)]
    ).reshape(NB_TC, 1, B)
    partials_sc = _phase1(node_feature, ids, zeros)
    partial_tc = _tc_partial(node_feature, ids_tc)
    return _combine(partials_sc.reshape(NC, G, D), partial_tc)


# speedup vs baseline: 1.1367x; 1.0371x over previous
import functools

import jax
import jax.numpy as jnp
from jax import lax
from jax.experimental import pallas as pl
from jax.experimental.pallas import tpu as pltpu
from jax.experimental.pallas import tpu_sc as plsc

N = 50000
D = 256
G = 128
NC = 2
NS = 16
NW = NC * NS
S = 80
L = 16

NSC = 7680
RTC = N - NSC
C = NSC // NW
NIT = C // S

B = 2048
NB_TC = -(-RTC // B)


def _phase1_body(x_hbm, ids_hbm, zeros_hbm, part_hbm,
                 buf_a, buf_b, buf_c, idx_a, idx_b, idx_c, acc, rbuf, slab,
                 sem_a, sem_b, sem_c, sem_ia, sem_ib, sem_ic, sem_z):
    cid = lax.axis_index("c")
    sid = lax.axis_index("s")
    wid = sid * NC + cid
    base = RTC + wid * C

    def start(i, buf, ibuf, rsem, isem):
        off = base + i * S
        pltpu.make_async_copy(x_hbm.at[pl.ds(off, S)], buf, rsem).start()
        pltpu.make_async_copy(ids_hbm.at[pl.ds(off, S)], ibuf, isem).start()

    start(0, buf_a, idx_a, sem_a, sem_ia)
    start(1, buf_b, idx_b, sem_b, sem_ib)
    start(2, buf_c, idx_c, sem_c, sem_ic)
    pltpu.make_async_copy(zeros_hbm, acc, sem_z).start()
    pltpu.make_async_copy(zeros_hbm, acc, sem_z).wait()

    bufs = ((buf_a, idx_a, sem_a, sem_ia), (buf_b, idx_b, sem_b, sem_ib),
            (buf_c, idx_c, sem_c, sem_ic))
    NB = len(bufs)

    def body(k, carry):
        for b in range(NB):
            buf, ibuf, rsem, isem = bufs[b]
            i = NB * k + b

            @pl.when(i < NIT)
            def _():
                off = base + i * S
                pltpu.make_async_copy(
                    x_hbm.at[pl.ds(off, S)], buf, rsem).wait()
                pltpu.make_async_copy(
                    ids_hbm.at[pl.ds(off, S)], ibuf, isem).wait()

                def group_body(g, gcarry):
                    idv = ibuf[pl.ds(g * L, L)]
                    seg0 = idv[0]
                    uniform = seg0 == idv[L - 1]

                    @pl.when(uniform)
                    def _():
                        gbase = seg0 * D
                        for j in range(D // L):
                            vs = [buf[g * L + r, pl.ds(L * j, L)]
                                  for r in range(L)]
                            while len(vs) > 1:
                                vs = [vs[t] + vs[t + 1]
                                      for t in range(0, len(vs), 2)]
                            plsc.addupdate(acc.at[pl.ds(gbase + L * j, L)],
                                           vs[0])

                    @pl.when(jnp.logical_not(uniform))
                    def _():
                        for r in range(L):
                            rbase = idv[r] * D
                            for j in range(D // L):
                                v = buf[g * L + r, pl.ds(L * j, L)]
                                plsc.addupdate(
                                    acc.at[pl.ds(rbase + L * j, L)], v)

                    return gcarry

                lax.fori_loop(0, S // L, group_body, 0)

                @pl.when(i + NB < NIT)
                def _():
                    start(i + NB, buf, ibuf, rsem, isem)
        return carry

    lax.fori_loop(0, (NIT + NB - 1) // NB, body, 0)

    R = 4
    GR = G // R
    W = (GR // NS) * D
    for h in range(R):
        pltpu.sync_copy(acc.at[pl.ds(h * GR * D, GR * D)],
                        slab.at[pl.ds(sid * GR * D, GR * D)])
        plsc.subcore_barrier()
        for s in range(NS):
            pltpu.sync_copy(slab.at[pl.ds(s * GR * D + sid * W, W)],
                            rbuf.at[pl.ds(s * W, W)])

        def red_body(w, rcarry):
            vs = [rbuf[pl.ds(s * W + w * L, L)] for s in range(NS)]
            while len(vs) > 1:
                vs = [vs[t] + vs[t + 1] for t in range(0, len(vs), 2)]
            rbuf[pl.ds(w * L, L)] = vs[0]
            return rcarry

        lax.fori_loop(0, W // L, red_body, 0)
        pltpu.sync_copy(
            rbuf.at[pl.ds(0, W)],
            part_hbm.at[pl.ds(cid * G * D + h * GR * D + sid * W, W)])
        plsc.subcore_barrier()


_phase1 = functools.partial(
    pl.kernel,
    out_type=jax.ShapeDtypeStruct((NC * G * D,), jnp.float32),
    mesh=plsc.VectorSubcoreMesh(core_axis_name="c", subcore_axis_name="s"),
    scratch_types=[
        pltpu.VMEM((S, D), jnp.float32),
        pltpu.VMEM((S, D), jnp.float32),
        pltpu.VMEM((S, D), jnp.float32),
        pltpu.VMEM((S,), jnp.int32),
        pltpu.VMEM((S,), jnp.int32),
        pltpu.VMEM((S,), jnp.int32),
        pltpu.VMEM((G * D,), jnp.float32),
        pltpu.VMEM((NS * (G // 4 // NS) * D,), jnp.float32),
        pltpu.MemorySpace.VMEM_SHARED((NS * (G // 4) * D,), jnp.float32),
        pltpu.SemaphoreType.DMA,
        pltpu.SemaphoreType.DMA,
        pltpu.SemaphoreType.DMA,
        pltpu.SemaphoreType.DMA,
        pltpu.SemaphoreType.DMA,
        pltpu.SemaphoreType.DMA,
        pltpu.SemaphoreType.DMA,
    ],
)(_phase1_body)


def _tc_body(x_ref, ids_ref, o_ref):
    ib = pl.program_id(0)

    @pl.when(ib == 0)
    def _():
        o_ref[...] = jnp.zeros((G, D), jnp.float32)

    ids_b = ids_ref[0, 0, :]
    onehot = (ids_b[:, None]
              == lax.broadcasted_iota(jnp.int32, (B, G), 1)).astype(jnp.bfloat16)
    x = x_ref[...]
    x_hi = x.astype(jnp.bfloat16)
    x_lo = (x - x_hi.astype(jnp.float32)).astype(jnp.bfloat16)
    dn = (((0,), (0,)), ((), ()))
    o_ref[...] += (
        lax.dot_general(onehot, x_hi, dn, preferred_element_type=jnp.float32)
        + lax.dot_general(onehot, x_lo, dn, preferred_element_type=jnp.float32))


def _tc_partial(x, ids_tc):
    return pl.pallas_call(
        _tc_body,
        grid=(NB_TC,),
        in_specs=[
            pl.BlockSpec((B, D), lambda ib: (ib, 0)),
            pl.BlockSpec((1, 1, B), lambda ib: (ib, 0, 0)),
        ],
        out_specs=pl.BlockSpec((G, D), lambda ib: (0, 0)),
        out_shape=jax.ShapeDtypeStruct((G, D), jnp.float32),
    )(x, ids_tc)


def _combine_body(p_ref, t_ref, o_ref):
    o_ref[...] = p_ref[0] + p_ref[1] + t_ref[...]


def _combine(partials_sc, partial_tc):
    return pl.pallas_call(
        _combine_body,
        out_shape=jax.ShapeDtypeStruct((G, D), jnp.float32),
    )(partials_sc, partial_tc)


@jax.jit
def kernel(node_feature, segment_ids, num_graphs):
    ids = segment_ids.astype(jnp.int32)
    zeros = jnp.zeros((G * D,), jnp.float32)
    ids_tc = jnp.concatenate(
        [ids[:RTC], jnp.full((NB_TC * B - RTC,), G, jnp.int32)]
    ).reshape(NB_TC, 1, B)
    partials_sc = _phase1(node_feature, ids, zeros)
    partial_tc = _tc_partial(node_feature, ids_tc)
    return _combine(partials_sc.reshape(NC, G, D), partial_tc)


# speedup vs baseline: 1.1914x; 1.0482x over previous
import functools

import jax
import jax.numpy as jnp
from jax import lax
from jax.experimental import pallas as pl
from jax.experimental.pallas import tpu as pltpu
from jax.experimental.pallas import tpu_sc as plsc

N = 50000
D = 256
G = 128
NC = 2
NS = 16
NW = NC * NS
S = 80
L = 16

NSC = 5120
RTC = N - NSC
C = NSC // NW
NIT = C // S

B = 2048
NB_TC = -(-RTC // B)


def _phase1_body(x_hbm, ids_hbm, zeros_hbm, part_hbm,
                 buf_a, buf_b, buf_c, idx_a, idx_b, idx_c, acc, rbuf, slab,
                 sem_a, sem_b, sem_c, sem_ia, sem_ib, sem_ic, sem_z):
    cid = lax.axis_index("c")
    sid = lax.axis_index("s")
    wid = sid * NC + cid
    base = RTC + wid * C

    def start(i, buf, ibuf, rsem, isem):
        off = base + i * S
        pltpu.make_async_copy(x_hbm.at[pl.ds(off, S)], buf, rsem).start()
        pltpu.make_async_copy(ids_hbm.at[pl.ds(off, S)], ibuf, isem).start()

    prime = ((buf_a, idx_a, sem_a, sem_ia), (buf_b, idx_b, sem_b, sem_ib),
             (buf_c, idx_c, sem_c, sem_ic))
    for i in range(min(3, NIT)):
        start(i, *prime[i])
    pltpu.make_async_copy(zeros_hbm, acc, sem_z).start()
    pltpu.make_async_copy(zeros_hbm, acc, sem_z).wait()

    bufs = ((buf_a, idx_a, sem_a, sem_ia), (buf_b, idx_b, sem_b, sem_ib),
            (buf_c, idx_c, sem_c, sem_ic))
    NB = len(bufs)

    def body(k, carry):
        for b in range(NB):
            buf, ibuf, rsem, isem = bufs[b]
            i = NB * k + b

            @pl.when(i < NIT)
            def _():
                off = base + i * S
                pltpu.make_async_copy(
                    x_hbm.at[pl.ds(off, S)], buf, rsem).wait()
                pltpu.make_async_copy(
                    ids_hbm.at[pl.ds(off, S)], ibuf, isem).wait()

                def group_body(g, gcarry):
                    idv = ibuf[pl.ds(g * L, L)]
                    seg0 = idv[0]
                    uniform = seg0 == idv[L - 1]

                    @pl.when(uniform)
                    def _():
                        gbase = seg0 * D
                        for j in range(D // L):
                            vs = [buf[g * L + r, pl.ds(L * j, L)]
                                  for r in range(L)]
                            while len(vs) > 1:
                                vs = [vs[t] + vs[t + 1]
                                      for t in range(0, len(vs), 2)]
                            plsc.addupdate(acc.at[pl.ds(gbase + L * j, L)],
                                           vs[0])

                    @pl.when(jnp.logical_not(uniform))
                    def _():
                        for r in range(L):
                            rbase = idv[r] * D
                            for j in range(D // L):
                                v = buf[g * L + r, pl.ds(L * j, L)]
                                plsc.addupdate(
                                    acc.at[pl.ds(rbase + L * j, L)], v)

                    return gcarry

                lax.fori_loop(0, S // L, group_body, 0)

                @pl.when(i + NB < NIT)
                def _():
                    start(i + NB, buf, ibuf, rsem, isem)
        return carry

    lax.fori_loop(0, (NIT + NB - 1) // NB, body, 0)

    R = 4
    GR = G // R
    W = (GR // NS) * D
    for h in range(R):
        pltpu.sync_copy(acc.at[pl.ds(h * GR * D, GR * D)],
                        slab.at[pl.ds(sid * GR * D, GR * D)])
        plsc.subcore_barrier()
        for s in range(NS):
            pltpu.sync_copy(slab.at[pl.ds(s * GR * D + sid * W, W)],
                            rbuf.at[pl.ds(s * W, W)])

        def red_body(w, rcarry):
            vs = [rbuf[pl.ds(s * W + w * L, L)] for s in range(NS)]
            while len(vs) > 1:
                vs = [vs[t] + vs[t + 1] for t in range(0, len(vs), 2)]
            rbuf[pl.ds(w * L, L)] = vs[0]
            return rcarry

        lax.fori_loop(0, W // L, red_body, 0)
        pltpu.sync_copy(
            rbuf.at[pl.ds(0, W)],
            part_hbm.at[pl.ds(cid * G * D + h * GR * D + sid * W, W)])
        plsc.subcore_barrier()


_phase1 = functools.partial(
    pl.kernel,
    out_type=jax.ShapeDtypeStruct((NC * G * D,), jnp.float32),
    mesh=plsc.VectorSubcoreMesh(core_axis_name="c", subcore_axis_name="s"),
    scratch_types=[
        pltpu.VMEM((S, D), jnp.float32),
        pltpu.VMEM((S, D), jnp.float32),
        pltpu.VMEM((S, D), jnp.float32),
        pltpu.VMEM((S,), jnp.int32),
        pltpu.VMEM((S,), jnp.int32),
        pltpu.VMEM((S,), jnp.int32),
        pltpu.VMEM((G * D,), jnp.float32),
        pltpu.VMEM((NS * (G // 4 // NS) * D,), jnp.float32),
        pltpu.MemorySpace.VMEM_SHARED((NS * (G // 4) * D,), jnp.float32),
        pltpu.SemaphoreType.DMA,
        pltpu.SemaphoreType.DMA,
        pltpu.SemaphoreType.DMA,
        pltpu.SemaphoreType.DMA,
        pltpu.SemaphoreType.DMA,
        pltpu.SemaphoreType.DMA,
        pltpu.SemaphoreType.DMA,
    ],
)(_phase1_body)


def _tc_body(x_ref, ids_ref, o_ref):
    ib = pl.program_id(0)

    @pl.when(ib == 0)
    def _():
        o_ref[...] = jnp.zeros((G, D), jnp.float32)

    ids_b = ids_ref[0, 0, :]
    onehot = (ids_b[:, None]
              == lax.broadcasted_iota(jnp.int32, (B, G), 1)).astype(jnp.bfloat16)
    x = x_ref[...]
    x_hi = x.astype(jnp.bfloat16)
    x_lo = (x - x_hi.astype(jnp.float32)).astype(jnp.bfloat16)
    dn = (((0,), (0,)), ((), ()))
    o_ref[...] += (
        lax.dot_general(onehot, x_hi, dn, preferred_element_type=jnp.float32)
        + lax.dot_general(onehot, x_lo, dn, preferred_element_type=jnp.float32))


def _tc_partial(x, ids_tc):
    return pl.pallas_call(
        _tc_body,
        grid=(NB_TC,),
        in_specs=[
            pl.BlockSpec((B, D), lambda ib: (ib, 0)),
            pl.BlockSpec((1, 1, B), lambda ib: (ib, 0, 0)),
        ],
        out_specs=pl.BlockSpec((G, D), lambda ib: (0, 0)),
        out_shape=jax.ShapeDtypeStruct((G, D), jnp.float32),
    )(x, ids_tc)


def _combine_body(p_ref, t_ref, o_ref):
    o_ref[...] = p_ref[0] + p_ref[1] + t_ref[...]


def _combine(partials_sc, partial_tc):
    return pl.pallas_call(
        _combine_body,
        out_shape=jax.ShapeDtypeStruct((G, D), jnp.float32),
    )(partials_sc, partial_tc)


@jax.jit
def kernel(node_feature, segment_ids, num_graphs):
    ids = segment_ids.astype(jnp.int32)
    zeros = jnp.zeros((G * D,), jnp.float32)
    ids_tc = jnp.concatenate(
        [ids[:RTC], jnp.full((NB_TC * B - RTC,), G, jnp.int32)]
    ).reshape(NB_TC, 1, B)
    partials_sc = _phase1(node_feature, ids, zeros)
    partial_tc = _tc_partial(node_feature, ids_tc)
    return _combine(partials_sc.reshape(NC, G, D), partial_tc)


# speedup vs baseline: 1.2331x; 1.0350x over previous
import functools

import jax
import jax.numpy as jnp
from jax import lax
from jax.experimental import pallas as pl
from jax.experimental.pallas import tpu as pltpu
from jax.experimental.pallas import tpu_sc as plsc

N = 50000
D = 256
G = 128
NC = 2
NS = 16
NW = NC * NS
S = 80
L = 16

NSC = 5120
RTC = N - NSC
C = NSC // NW
NIT = C // S

B = 2048
NB_TC = -(-RTC // B)


def _phase1_body(x_hbm, ids_hbm, zeros_hbm, part_hbm,
                 buf_a, buf_b, buf_c, idx_a, idx_b, idx_c, acc, rbuf, slab,
                 sem_a, sem_b, sem_c, sem_ia, sem_ib, sem_ic, sem_z):
    cid = lax.axis_index("c")
    sid = lax.axis_index("s")
    wid = sid * NC + cid
    base = RTC + wid * C

    def start(i, buf, ibuf, rsem, isem):
        off = base + i * S
        pltpu.make_async_copy(x_hbm.at[pl.ds(off, S)], buf, rsem).start()
        pltpu.make_async_copy(ids_hbm.at[pl.ds(off, S)], ibuf, isem).start()

    prime = ((buf_a, idx_a, sem_a, sem_ia), (buf_b, idx_b, sem_b, sem_ib),
             (buf_c, idx_c, sem_c, sem_ic))
    for i in range(min(3, NIT)):
        start(i, *prime[i])
    pltpu.make_async_copy(zeros_hbm, acc, sem_z).start()
    pltpu.make_async_copy(zeros_hbm, acc, sem_z).wait()

    bufs = ((buf_a, idx_a, sem_a, sem_ia), (buf_b, idx_b, sem_b, sem_ib),
            (buf_c, idx_c, sem_c, sem_ic))
    NB = len(bufs)

    def body(k, carry):
        for b in range(NB):
            buf, ibuf, rsem, isem = bufs[b]
            i = NB * k + b

            @pl.when(i < NIT)
            def _():
                off = base + i * S
                pltpu.make_async_copy(
                    x_hbm.at[pl.ds(off, S)], buf, rsem).wait()
                pltpu.make_async_copy(
                    ids_hbm.at[pl.ds(off, S)], ibuf, isem).wait()

                def group_body(g, gcarry):
                    idv = ibuf[pl.ds(g * L, L)]
                    seg0 = idv[0]
                    uniform = seg0 == idv[L - 1]

                    @pl.when(uniform)
                    def _():
                        for j in range(D // L):
                            vs = [buf[g * L + r, pl.ds(L * j, L)]
                                  for r in range(L)]
                            while len(vs) > 1:
                                vs = [vs[t] + vs[t + 1]
                                      for t in range(0, len(vs), 2)]
                            plsc.addupdate(acc.at[seg0, pl.ds(L * j, L)],
                                           vs[0])

                    @pl.when(jnp.logical_not(uniform))
                    def _():
                        for r in range(L):
                            seg = idv[r]
                            for j in range(D // L):
                                v = buf[g * L + r, pl.ds(L * j, L)]
                                plsc.addupdate(
                                    acc.at[seg, pl.ds(L * j, L)], v)

                    return gcarry

                lax.fori_loop(0, S // L, group_body, 0)

                @pl.when(i + NB < NIT)
                def _():
                    start(i + NB, buf, ibuf, rsem, isem)
        return carry

    lax.fori_loop(0, (NIT + NB - 1) // NB, body, 0)

    R = 4
    GR = G // R
    GS = GR // NS
    for h in range(R):
        pltpu.sync_copy(acc.at[pl.ds(h * GR, GR), :],
                        slab.at[pl.ds(sid * GR, GR), :])
        plsc.subcore_barrier()
        for s in range(NS):
            pltpu.sync_copy(slab.at[pl.ds(s * GR + sid * GS, GS), :],
                            rbuf.at[pl.ds(s * GS, GS), :])

        def red_body(w, rcarry):
            q = w >> 4
            cc = (w & (L - 1)) * L
            vs = [rbuf[s * GS + q, pl.ds(cc, L)] for s in range(NS)]
            while len(vs) > 1:
                vs = [vs[t] + vs[t + 1] for t in range(0, len(vs), 2)]
            rbuf[q, pl.ds(cc, L)] = vs[0]
            return rcarry

        lax.fori_loop(0, GS * D // L, red_body, 0)
        pltpu.sync_copy(
            rbuf.at[pl.ds(0, GS), :],
            part_hbm.at[pl.ds(cid * G + h * GR + sid * GS, GS), :])
        plsc.subcore_barrier()


_phase1 = functools.partial(
    pl.kernel,
    out_type=jax.ShapeDtypeStruct((NC * G, D), jnp.float32),
    mesh=plsc.VectorSubcoreMesh(core_axis_name="c", subcore_axis_name="s"),
    scratch_types=[
        pltpu.VMEM((S, D), jnp.float32),
        pltpu.VMEM((S, D), jnp.float32),
        pltpu.VMEM((S, D), jnp.float32),
        pltpu.VMEM((S,), jnp.int32),
        pltpu.VMEM((S,), jnp.int32),
        pltpu.VMEM((S,), jnp.int32),
        pltpu.VMEM((G, D), jnp.float32),
        pltpu.VMEM((NS * (G // 4 // NS), D), jnp.float32),
        pltpu.MemorySpace.VMEM_SHARED((NS * (G // 4), D), jnp.float32),
        pltpu.SemaphoreType.DMA,
        pltpu.SemaphoreType.DMA,
        pltpu.SemaphoreType.DMA,
        pltpu.SemaphoreType.DMA,
        pltpu.SemaphoreType.DMA,
        pltpu.SemaphoreType.DMA,
        pltpu.SemaphoreType.DMA,
    ],
)(_phase1_body)


def _tc_body(x_ref, ids_ref, o_ref):
    ib = pl.program_id(0)

    @pl.when(ib == 0)
    def _():
        o_ref[...] = jnp.zeros((G, D), jnp.float32)

    ids_b = ids_ref[0, 0, :]
    onehot = (ids_b[:, None]
              == lax.broadcasted_iota(jnp.int32, (B, G), 1)).astype(jnp.bfloat16)
    x = x_ref[...]
    x_hi = x.astype(jnp.bfloat16)
    x_lo = (x - x_hi.astype(jnp.float32)).astype(jnp.bfloat16)
    dn = (((0,), (0,)), ((), ()))
    o_ref[...] += (
        lax.dot_general(onehot, x_hi, dn, preferred_element_type=jnp.float32)
        + lax.dot_general(onehot, x_lo, dn, preferred_element_type=jnp.float32))


def _tc_partial(x, ids_tc):
    return pl.pallas_call(
        _tc_body,
        grid=(NB_TC,),
        in_specs=[
            pl.BlockSpec((B, D), lambda ib: (ib, 0)),
            pl.BlockSpec((1, 1, B), lambda ib: (ib, 0, 0)),
        ],
        out_specs=pl.BlockSpec((G, D), lambda ib: (0, 0)),
        out_shape=jax.ShapeDtypeStruct((G, D), jnp.float32),
    )(x, ids_tc)


def _combine_body(p_ref, t_ref, o_ref):
    o_ref[...] = p_ref[pl.ds(0, G), :] + p_ref[pl.ds(G, G), :] + t_ref[...]


def _combine(partials_sc, partial_tc):
    return pl.pallas_call(
        _combine_body,
        out_shape=jax.ShapeDtypeStruct((G, D), jnp.float32),
    )(partials_sc, partial_tc)


@jax.jit
def kernel(node_feature, segment_ids, num_graphs):
    ids = segment_ids.astype(jnp.int32)
    zeros = jnp.zeros((G, D), jnp.float32)
    ids_tc = jnp.concatenate(
        [ids[:RTC], jnp.full((NB_TC * B - RTC,), G, jnp.int32)]
    ).reshape(NB_TC, 1, B)
    partials_sc = _phase1(node_feature, ids, zeros)
    partial_tc = _tc_partial(node_feature, ids_tc)
    return _combine(partials_sc, partial_tc)


# speedup vs baseline: 1.2505x; 1.0142x over previous
import functools

import jax
import jax.numpy as jnp
from jax import lax
from jax.experimental import pallas as pl
from jax.experimental.pallas import tpu as pltpu
from jax.experimental.pallas import tpu_sc as plsc

N = 50000
D = 256
G = 128
NC = 2
NS = 16
NW = NC * NS
S = 80
L = 16

NSC = 5120
RTC = N - NSC
C = NSC // NW
NIT = C // S

B = 4096
NB_TC = -(-RTC // B)


def _phase1_body(x_hbm, ids_hbm, zeros_hbm, part_hbm,
                 buf_a, buf_b, buf_c, idx_a, idx_b, idx_c, acc, rbuf, slab,
                 sem_a, sem_b, sem_c, sem_ia, sem_ib, sem_ic, sem_z):
    cid = lax.axis_index("c")
    sid = lax.axis_index("s")
    wid = sid * NC + cid
    base = RTC + wid * C

    def start(i, buf, ibuf, rsem, isem):
        off = base + i * S
        pltpu.make_async_copy(x_hbm.at[pl.ds(off, S)], buf, rsem).start()
        pltpu.make_async_copy(ids_hbm.at[pl.ds(off, S)], ibuf, isem).start()

    prime = ((buf_a, idx_a, sem_a, sem_ia), (buf_b, idx_b, sem_b, sem_ib),
             (buf_c, idx_c, sem_c, sem_ic))
    for i in range(min(3, NIT)):
        start(i, *prime[i])
    pltpu.make_async_copy(zeros_hbm, acc, sem_z).start()
    pltpu.make_async_copy(zeros_hbm, acc, sem_z).wait()

    bufs = ((buf_a, idx_a, sem_a, sem_ia), (buf_b, idx_b, sem_b, sem_ib),
            (buf_c, idx_c, sem_c, sem_ic))
    NB = len(bufs)

    def body(k, carry):
        for b in range(NB):
            buf, ibuf, rsem, isem = bufs[b]
            i = NB * k + b

            @pl.when(i < NIT)
            def _():
                off = base + i * S
                pltpu.make_async_copy(
                    x_hbm.at[pl.ds(off, S)], buf, rsem).wait()
                pltpu.make_async_copy(
                    ids_hbm.at[pl.ds(off, S)], ibuf, isem).wait()

                def group_body(g, gcarry):
                    idv = ibuf[pl.ds(g * L, L)]
                    seg0 = idv[0]
                    uniform = seg0 == idv[L - 1]

                    @pl.when(uniform)
                    def _():
                        for j in range(D // L):
                            vs = [buf[g * L + r, pl.ds(L * j, L)]
                                  for r in range(L)]
                            while len(vs) > 1:
                                vs = [vs[t] + vs[t + 1]
                                      for t in range(0, len(vs), 2)]
                            plsc.addupdate(acc.at[seg0, pl.ds(L * j, L)],
                                           vs[0])

                    @pl.when(jnp.logical_not(uniform))
                    def _():
                        for r in range(L):
                            seg = idv[r]
                            for j in range(D // L):
                                v = buf[g * L + r, pl.ds(L * j, L)]
                                plsc.addupdate(
                                    acc.at[seg, pl.ds(L * j, L)], v)

                    return gcarry

                lax.fori_loop(0, S // L, group_body, 0)

                @pl.when(i + NB < NIT)
                def _():
                    start(i + NB, buf, ibuf, rsem, isem)
        return carry

    lax.fori_loop(0, (NIT + NB - 1) // NB, body, 0)

    R = 4
    GR = G // R
    GS = GR // NS
    for h in range(R):
        pltpu.sync_copy(acc.at[pl.ds(h * GR, GR), :],
                        slab.at[pl.ds(sid * GR, GR), :])
        plsc.subcore_barrier()
        for s in range(NS):
            pltpu.sync_copy(slab.at[pl.ds(s * GR + sid * GS, GS), :],
                            rbuf.at[pl.ds(s * GS, GS), :])

        def red_body(w, rcarry):
            q = w >> 4
            cc = (w & (L - 1)) * L
            vs = [rbuf[s * GS + q, pl.ds(cc, L)] for s in range(NS)]
            while len(vs) > 1:
                vs = [vs[t] + vs[t + 1] for t in range(0, len(vs), 2)]
            rbuf[q, pl.ds(cc, L)] = vs[0]
            return rcarry

        lax.fori_loop(0, GS * D // L, red_body, 0)
        pltpu.sync_copy(
            rbuf.at[pl.ds(0, GS), :],
            part_hbm.at[pl.ds(cid * G + h * GR + sid * GS, GS), :])
        plsc.subcore_barrier()


_phase1 = functools.partial(
    pl.kernel,
    out_type=jax.ShapeDtypeStruct((NC * G, D), jnp.float32),
    mesh=plsc.VectorSubcoreMesh(core_axis_name="c", subcore_axis_name="s"),
    scratch_types=[
        pltpu.VMEM((S, D), jnp.float32),
        pltpu.VMEM((S, D), jnp.float32),
        pltpu.VMEM((S, D), jnp.float32),
        pltpu.VMEM((S,), jnp.int32),
        pltpu.VMEM((S,), jnp.int32),
        pltpu.VMEM((S,), jnp.int32),
        pltpu.VMEM((G, D), jnp.float32),
        pltpu.VMEM((NS * (G // 4 // NS), D), jnp.float32),
        pltpu.MemorySpace.VMEM_SHARED((NS * (G // 4), D), jnp.float32),
        pltpu.SemaphoreType.DMA,
        pltpu.SemaphoreType.DMA,
        pltpu.SemaphoreType.DMA,
        pltpu.SemaphoreType.DMA,
        pltpu.SemaphoreType.DMA,
        pltpu.SemaphoreType.DMA,
        pltpu.SemaphoreType.DMA,
    ],
)(_phase1_body)


def _tc_body(x_ref, ids_ref, o_ref):
    ib = pl.program_id(0)

    @pl.when(ib == 0)
    def _():
        o_ref[...] = jnp.zeros((G, D), jnp.float32)

    ids_b = ids_ref[0, 0, :]
    onehot = (ids_b[:, None]
              == lax.broadcasted_iota(jnp.int32, (B, G), 1)).astype(jnp.bfloat16)
    x = x_ref[...]
    x_hi = x.astype(jnp.bfloat16)
    x_lo = (x - x_hi.astype(jnp.float32)).astype(jnp.bfloat16)
    dn = (((0,), (0,)), ((), ()))
    o_ref[...] += (
        lax.dot_general(onehot, x_hi, dn, preferred_element_type=jnp.float32)
        + lax.dot_general(onehot, x_lo, dn, preferred_element_type=jnp.float32))


def _tc_partial(x, ids_tc):
    return pl.pallas_call(
        _tc_body,
        grid=(NB_TC,),
        in_specs=[
            pl.BlockSpec((B, D), lambda ib: (ib, 0)),
            pl.BlockSpec((1, 1, B), lambda ib: (ib, 0, 0)),
        ],
        out_specs=pl.BlockSpec((G, D), lambda ib: (0, 0)),
        out_shape=jax.ShapeDtypeStruct((G, D), jnp.float32),
    )(x, ids_tc)


def _combine_body(p_ref, t_ref, o_ref):
    o_ref[...] = p_ref[pl.ds(0, G), :] + p_ref[pl.ds(G, G), :] + t_ref[...]


def _combine(partials_sc, partial_tc):
    return pl.pallas_call(
        _combine_body,
        out_shape=jax.ShapeDtypeStruct((G, D), jnp.float32),
    )(partials_sc, partial_tc)


@jax.jit
def kernel(node_feature, segment_ids, num_graphs):
    ids = segment_ids.astype(jnp.int32)
    zeros = jnp.zeros((G, D), jnp.float32)
    ids_tc = jnp.concatenate(
        [ids[:RTC], jnp.full((NB_TC * B - RTC,), G, jnp.int32)]
    ).reshape(NB_TC, 1, B)
    partials_sc = _phase1(node_feature, ids, zeros)
    partial_tc = _tc_partial(node_feature, ids_tc)
    return _combine(partials_sc, partial_tc)
